# baseline (speedup 1.0000x reference)
import jax
import jax.numpy as jnp
from jax.experimental import pallas as pl
from jax.experimental.pallas import tpu as pltpu

D_MODEL_ = 4096
N_EXPERTS_ = 8
N_MODULES_ = 4
K_TOP_ = 2
B_ = 8192

B_T = 2048
C_J = 1024
C_K = 512
N_BT = B_ // B_T
N_J = D_MODEL_ // C_J
N_K = D_MODEL_ // C_K

_PREC = jax.lax.Precision.DEFAULT


def _router_kernel(p_ref, wg_ref, wr_ref, out_ref, gated_ref, acc_ref):
    j = pl.program_id(1)
    k = pl.program_id(2)
    sl = pl.ds(k * C_K, C_K)

    part = jax.lax.dot_general(
        p_ref[...], wg_ref[...], (((1,), (1,)), ((), ())),
        precision=_PREC, preferred_element_type=jnp.float32)

    @pl.when(j == 0)
    def _():
        gated_ref[:, sl] = part

    @pl.when(j > 0)
    def _():
        gated_ref[:, sl] += part

    @pl.when(j == N_J - 1)
    def _():
        @pl.when(k == 0)
        def _():
            acc_ref[...] = jnp.zeros_like(acc_ref)

        acc_ref[...] += jax.lax.dot_general(
            gated_ref[:, sl], wr_ref[...], (((1,), (1,)), ((), ())),
            precision=_PREC, preferred_element_type=jnp.float32)

        @pl.when(k == N_K - 1)
        def _():
            logits = acc_ref[...]
            m = jnp.max(logits, axis=-1, keepdims=True)
            e = jnp.exp(logits - m)
            denom = jnp.sum(e, axis=-1, keepdims=True)
            probs = e / denom
            col = jax.lax.broadcasted_iota(
                jnp.int32, (B_T, N_EXPERTS_), 1)
            hi = jnp.where(col < K_TOP_, 1.0 / K_TOP_, 0.0).astype(jnp.float32)
            lo = jnp.where(col < 1, 1.0, 0.0).astype(jnp.float32)
            masks = [jnp.where(probs[:, i:i + 1] > 0.5, hi, lo)
                     for i in range(N_MODULES_)]
            out_ref[...] = jnp.concatenate(masks, axis=1)


def _make_call():
    out_spec = pl.BlockSpec((B_T, N_MODULES_ * N_EXPERTS_),
                            lambda i, j, k: (i, 0))
    return pl.pallas_call(
        _router_kernel,
        grid=(N_BT, N_J, N_K),
        in_specs=[
            pl.BlockSpec((B_T, C_J), lambda i, j, k: (i, j)),
            pl.BlockSpec((C_K, C_J), lambda i, j, k: (k, j)),
            pl.BlockSpec((N_MODULES_, C_K), lambda i, j, k: (0, k)),
        ],
        out_specs=out_spec,
        out_shape=jax.ShapeDtypeStruct(
            (B_, N_MODULES_ * N_EXPERTS_), jnp.float32),
        scratch_shapes=[
            pltpu.VMEM((B_T, D_MODEL_), jnp.float32),
            pltpu.VMEM((B_T, N_MODULES_), jnp.float32),
        ],
        compiler_params=pltpu.CompilerParams(
            dimension_semantics=("parallel", "arbitrary", "arbitrary"),
            vmem_limit_bytes=100 * 1024 * 1024,
        ),
    )


def kernel(pooled_hidden, Wg, Wr):
    out = _make_call()(pooled_hidden, Wg, Wr)
    return tuple(out[:, i * N_EXPERTS_:(i + 1) * N_EXPERTS_]
                 for i in range(N_MODULES_))


# speedup vs baseline: 1.2054x; 1.2054x over previous
import jax
import jax.numpy as jnp
from jax.experimental import pallas as pl
from jax.experimental.pallas import tpu as pltpu

D_MODEL_ = 4096
N_EXPERTS_ = 8
N_MODULES_ = 4
K_TOP_ = 2
B_ = 8192

B_T = 1024
C_G = 512
N_BT = B_ // B_T
N_KC = D_MODEL_ // C_G


def _router_kernel(p_ref, wg_ref, wr_ref, q_ref, k_ref, v_ref, o_ref,
                   acc_ref):
    kc = pl.program_id(1)

    @pl.when(kc == 0)
    def _():
        acc_ref[...] = jnp.zeros_like(acc_ref)

    gated = jax.lax.dot_general(
        p_ref[...], wg_ref[...], (((1,), (1,)), ((), ())),
        preferred_element_type=jnp.float32)
    acc_ref[...] += jax.lax.dot_general(
        gated, wr_ref[...], (((1,), (1,)), ((), ())),
        preferred_element_type=jnp.float32)

    @pl.when(kc == N_KC - 1)
    def _():
        logits = acc_ref[...]
        m = jnp.max(logits, axis=-1, keepdims=True)
        e = jnp.exp(logits - m)
        probs = e / jnp.sum(e, axis=-1, keepdims=True)
        col = jax.lax.broadcasted_iota(jnp.int32, (B_T, N_EXPERTS_), 1)
        hi = jnp.where(col < K_TOP_, 1.0 / K_TOP_, 0.0).astype(jnp.float32)
        lo = jnp.where(col < 1, 1.0, 0.0).astype(jnp.float32)
        for i, ref in enumerate((q_ref, k_ref, v_ref, o_ref)):
            sel = probs[:, i:i + 1] > 0.5
            ref[...] = jnp.where(sel, hi, lo)


def _make_call():
    out_spec = pl.BlockSpec((B_T, N_EXPERTS_), lambda i, k: (i, 0))
    return pl.pallas_call(
        _router_kernel,
        grid=(N_BT, N_KC),
        in_specs=[
            pl.BlockSpec((B_T, D_MODEL_), lambda i, k: (i, 0)),
            pl.BlockSpec((C_G, D_MODEL_), lambda i, k: (k, 0)),
            pl.BlockSpec((N_MODULES_, C_G), lambda i, k: (0, k)),
        ],
        out_specs=[out_spec] * N_MODULES_,
        out_shape=[jax.ShapeDtypeStruct((B_, N_EXPERTS_), jnp.float32)] * N_MODULES_,
        scratch_shapes=[pltpu.VMEM((B_T, N_MODULES_), jnp.float32)],
        compiler_params=pltpu.CompilerParams(
            dimension_semantics=("parallel", "arbitrary"),
        ),
    )


def kernel(pooled_hidden, Wg, Wr):
    q, k, v, o = _make_call()(pooled_hidden, Wg, Wr)
    return (q, k, v, o)


# speedup vs baseline: 1.6391x; 1.3598x over previous
import jax
import jax.numpy as jnp
from jax.experimental import pallas as pl
from jax.experimental.pallas import tpu as pltpu

D_MODEL_ = 4096
N_EXPERTS_ = 8
N_MODULES_ = 4
K_TOP_ = 2
B_ = 8192

B_T = 1024
N_BT = B_ // B_T
C_K = 1024
N_KC = D_MODEL_ // C_K
C_G = 512
N_GC = D_MODEL_ // C_G
WINDOW = 0.0125
CAP = 1024


def _mask_rows(probs, n_rows):
    col = jax.lax.broadcasted_iota(jnp.int32, (n_rows, N_EXPERTS_), 1)
    hi = jnp.where(col < K_TOP_, 1.0 / K_TOP_, 0.0).astype(jnp.float32)
    lo = jnp.where(col < 1, 1.0, 0.0).astype(jnp.float32)
    outs = []
    for i in range(N_MODULES_):
        sel = probs[:, i:i + 1] > 0.5
        outs.append(jnp.where(sel, hi, lo))
    return outs


def _softmax4(logits):
    m = jnp.max(logits, axis=-1, keepdims=True)
    e = jnp.exp(logits - m)
    return e / jnp.sum(e, axis=-1, keepdims=True)


def _collapse_kernel(wr_ref, wg_ref, c_ref):
    k = pl.program_id(0)

    @pl.when(k == 0)
    def _():
        c_ref[...] = jnp.zeros_like(c_ref)

    c_ref[...] += jax.lax.dot_general(
        wr_ref[...], wg_ref[...], (((1,), (0,)), ((), ())),
        preferred_element_type=jnp.float32)


def _collapse_call():
    return pl.pallas_call(
        _collapse_kernel,
        grid=(N_KC,),
        in_specs=[
            pl.BlockSpec((N_MODULES_, C_K), lambda k: (0, k)),
            pl.BlockSpec((C_K, D_MODEL_), lambda k: (k, 0)),
        ],
        out_specs=pl.BlockSpec((N_MODULES_, D_MODEL_), lambda k: (0, 0)),
        out_shape=jax.ShapeDtypeStruct((N_MODULES_, D_MODEL_), jnp.float32),
        compiler_params=pltpu.CompilerParams(
            dimension_semantics=("arbitrary",),
        ),
    )


def _route_kernel(x_ref, c_ref, q_ref, k_ref, v_ref, o_ref, f_ref):
    logits = jax.lax.dot_general(
        x_ref[...], c_ref[...], (((1,), (1,)), ((), ())),
        preferred_element_type=jnp.float32)
    probs = _softmax4(logits)
    for ref, mask in zip((q_ref, k_ref, v_ref, o_ref),
                         _mask_rows(probs, B_T)):
        ref[...] = mask
    amb = jnp.any(jnp.abs(probs - 0.5) < WINDOW, axis=-1, keepdims=True)
    f_ref[...] = jnp.broadcast_to(amb, (B_T, N_EXPERTS_)).astype(jnp.int32)


def _route_call():
    out_spec = pl.BlockSpec((B_T, N_EXPERTS_), lambda i: (i, 0))
    return pl.pallas_call(
        _route_kernel,
        grid=(N_BT,),
        in_specs=[
            pl.BlockSpec((B_T, D_MODEL_), lambda i: (i, 0)),
            pl.BlockSpec((N_MODULES_, D_MODEL_), lambda i: (0, 0)),
        ],
        out_specs=[out_spec] * (N_MODULES_ + 1),
        out_shape=(
            [jax.ShapeDtypeStruct((B_, N_EXPERTS_), jnp.float32)] * N_MODULES_
            + [jax.ShapeDtypeStruct((B_, N_EXPERTS_), jnp.int32)]),
        compiler_params=pltpu.CompilerParams(
            dimension_semantics=("arbitrary",),
        ),
    )


def _faithful_kernel(p_ref, wg_ref, wr_ref, q_ref, k_ref, v_ref, o_ref,
                     acc_ref):
    kc = pl.program_id(1)

    @pl.when(kc == 0)
    def _():
        acc_ref[...] = jnp.zeros_like(acc_ref)

    gated = jax.lax.dot_general(
        p_ref[...], wg_ref[...], (((1,), (1,)), ((), ())),
        preferred_element_type=jnp.float32)
    acc_ref[...] += jax.lax.dot_general(
        gated, wr_ref[...], (((1,), (1,)), ((), ())),
        preferred_element_type=jnp.float32)

    @pl.when(kc == N_GC - 1)
    def _():
        probs = _softmax4(acc_ref[...])
        for ref, mask in zip((q_ref, k_ref, v_ref, o_ref),
                             _mask_rows(probs, p_ref.shape[0])):
            ref[...] = mask


def _faithful_call(n_rows):
    out_spec = pl.BlockSpec((B_T, N_EXPERTS_), lambda i, k: (i, 0))
    return pl.pallas_call(
        _faithful_kernel,
        grid=(n_rows // B_T, N_GC),
        in_specs=[
            pl.BlockSpec((B_T, D_MODEL_), lambda i, k: (i, 0)),
            pl.BlockSpec((C_G, D_MODEL_), lambda i, k: (k, 0)),
            pl.BlockSpec((N_MODULES_, C_G), lambda i, k: (0, k)),
        ],
        out_specs=[out_spec] * N_MODULES_,
        out_shape=[jax.ShapeDtypeStruct((n_rows, N_EXPERTS_), jnp.float32)]
        * N_MODULES_,
        scratch_shapes=[pltpu.VMEM((B_T, N_MODULES_), jnp.float32)],
        compiler_params=pltpu.CompilerParams(
            dimension_semantics=("parallel", "arbitrary"),
        ),
    )


def kernel(pooled_hidden, Wg, Wr):
    c = _collapse_call()(Wr, Wg)
    q, k, v, o, flags = _route_call()(pooled_hidden, c)
    flag_row = flags[:, 0]
    n_amb = jnp.sum(flag_row)
    idx = jnp.nonzero(flag_row, size=CAP, fill_value=0)[0].astype(jnp.int32)

    def hybrid():
        x_amb = pooled_hidden[idx]
        rq, rk, rv, ro = _faithful_call(CAP)(x_amb, Wg, Wr)
        return (q.at[idx].set(rq), k.at[idx].set(rk),
                v.at[idx].set(rv), o.at[idx].set(ro))

    def full_fallback():
        return tuple(_faithful_call(B_)(pooled_hidden, Wg, Wr))

    return jax.lax.cond(n_amb <= CAP, hybrid, full_fallback)


# speedup vs baseline: 2.4739x; 1.5093x over previous
import jax
import jax.numpy as jnp
from jax.experimental import pallas as pl
from jax.experimental.pallas import tpu as pltpu

D_MODEL_ = 4096
N_EXPERTS_ = 8
N_MODULES_ = 4
K_TOP_ = 2
B_ = 8192
M_W = N_MODULES_ * N_EXPERTS_

B_T = 1024
N_BT = B_ // B_T
C_K = 1024
N_KC = D_MODEL_ // C_K
C_G = 512
N_GC = D_MODEL_ // C_G
WINDOW = 0.01
CAP = 512


def _masks32(probs, n_rows):
    col = jax.lax.broadcasted_iota(jnp.int32, (n_rows, N_EXPERTS_), 1)
    hi = jnp.where(col < K_TOP_, 1.0 / K_TOP_, 0.0).astype(jnp.float32)
    lo = jnp.where(col < 1, 1.0, 0.0).astype(jnp.float32)
    parts = []
    for i in range(N_MODULES_):
        sel = probs[:, i:i + 1] > 0.5
        parts.append(jnp.where(sel, hi, lo))
    return jnp.concatenate(parts, axis=-1)


def _softmax4(logits):
    m = jnp.max(logits, axis=-1, keepdims=True)
    e = jnp.exp(logits - m)
    return e / jnp.sum(e, axis=-1, keepdims=True)


def _collapse_kernel(wr_ref, wg_ref, c_ref):
    k = pl.program_id(0)

    @pl.when(k == 0)
    def _():
        c_ref[...] = jnp.zeros_like(c_ref)

    c_ref[...] += jax.lax.dot_general(
        wr_ref[...], wg_ref[...], (((1,), (0,)), ((), ())),
        preferred_element_type=jnp.float32)


def _collapse_call():
    return pl.pallas_call(
        _collapse_kernel,
        grid=(N_KC,),
        in_specs=[
            pl.BlockSpec((N_MODULES_, C_K), lambda k: (0, k)),
            pl.BlockSpec((C_K, D_MODEL_), lambda k: (k, 0)),
        ],
        out_specs=pl.BlockSpec((N_MODULES_, D_MODEL_), lambda k: (0, 0)),
        out_shape=jax.ShapeDtypeStruct((N_MODULES_, D_MODEL_), jnp.float32),
        compiler_params=pltpu.CompilerParams(
            dimension_semantics=("arbitrary",),
        ),
    )


def _route_kernel(x_ref, c_ref, m_ref, f_ref):
    logits = jax.lax.dot_general(
        x_ref[...], c_ref[...], (((1,), (1,)), ((), ())),
        preferred_element_type=jnp.float32)
    probs = _softmax4(logits)
    m_ref[...] = _masks32(probs, B_T)
    amb = jnp.any(jnp.abs(probs - 0.5) < WINDOW, axis=-1, keepdims=True)
    f_ref[...] = jnp.broadcast_to(amb, (B_T, N_EXPERTS_)).astype(jnp.int32)


def _route_call():
    return pl.pallas_call(
        _route_kernel,
        grid=(N_BT,),
        in_specs=[
            pl.BlockSpec((B_T, D_MODEL_), lambda i: (i, 0)),
            pl.BlockSpec((N_MODULES_, D_MODEL_), lambda i: (0, 0)),
        ],
        out_specs=[
            pl.BlockSpec((B_T, M_W), lambda i: (i, 0)),
            pl.BlockSpec((B_T, N_EXPERTS_), lambda i: (i, 0)),
        ],
        out_shape=[
            jax.ShapeDtypeStruct((B_, M_W), jnp.float32),
            jax.ShapeDtypeStruct((B_, N_EXPERTS_), jnp.int32),
        ],
        compiler_params=pltpu.CompilerParams(
            dimension_semantics=("arbitrary",),
        ),
    )


def _faithful_kernel(p_ref, wg_ref, wr_ref, m_ref, acc_ref):
    kc = pl.program_id(1)

    @pl.when(kc == 0)
    def _():
        acc_ref[...] = jnp.zeros_like(acc_ref)

    gated = jax.lax.dot_general(
        p_ref[...], wg_ref[...], (((1,), (1,)), ((), ())),
        preferred_element_type=jnp.float32)
    acc_ref[...] += jax.lax.dot_general(
        gated, wr_ref[...], (((1,), (1,)), ((), ())),
        preferred_element_type=jnp.float32)

    @pl.when(kc == N_GC - 1)
    def _():
        probs = _softmax4(acc_ref[...])
        m_ref[...] = _masks32(probs, p_ref.shape[0])


def _faithful_call(n_rows):
    r_t = min(n_rows, B_T)
    return pl.pallas_call(
        _faithful_kernel,
        grid=(n_rows // r_t, N_GC),
        in_specs=[
            pl.BlockSpec((r_t, D_MODEL_), lambda i, k: (i, 0)),
            pl.BlockSpec((C_G, D_MODEL_), lambda i, k: (k, 0)),
            pl.BlockSpec((N_MODULES_, C_G), lambda i, k: (0, k)),
        ],
        out_specs=pl.BlockSpec((r_t, M_W), lambda i, k: (i, 0)),
        out_shape=jax.ShapeDtypeStruct((n_rows, M_W), jnp.float32),
        scratch_shapes=[pltpu.VMEM((r_t, N_MODULES_), jnp.float32)],
        compiler_params=pltpu.CompilerParams(
            dimension_semantics=("parallel", "arbitrary"),
        ),
    )


def kernel(pooled_hidden, Wg, Wr):
    c = _collapse_call()(Wr, Wg)
    masks, flags = _route_call()(pooled_hidden, c)
    flag_row = flags[:, 0]
    n_amb = jnp.sum(flag_row)
    idx = jnp.nonzero(flag_row, size=CAP, fill_value=0)[0].astype(jnp.int32)

    def hybrid():
        x_amb = pooled_hidden[idx]
        rm = _faithful_call(CAP)(x_amb, Wg, Wr)
        return masks.at[idx].set(rm)

    def full_fallback():
        return _faithful_call(B_)(pooled_hidden, Wg, Wr)

    m = jax.lax.cond(n_amb <= CAP, hybrid, full_fallback)
    return tuple(m[:, i * N_EXPERTS_:(i + 1) * N_EXPERTS_]
                 for i in range(N_MODULES_))


# speedup vs baseline: 2.5168x; 1.0173x over previous
import jax
import jax.numpy as jnp
from jax.experimental import pallas as pl
from jax.experimental.pallas import tpu as pltpu

D_MODEL_ = 4096
N_EXPERTS_ = 8
N_MODULES_ = 4
K_TOP_ = 2
B_ = 8192
M_W = N_MODULES_ * N_EXPERTS_

B_T = 1024
N_BT = B_ // B_T
C_K = 512
N_KC = D_MODEL_ // C_K
C_G = 512
N_GC = D_MODEL_ // C_G
WINDOW = 0.01
CAP = 512


def _masks32(probs, n_rows):
    col = jax.lax.broadcasted_iota(jnp.int32, (n_rows, N_EXPERTS_), 1)
    hi = jnp.where(col < K_TOP_, 1.0 / K_TOP_, 0.0).astype(jnp.float32)
    lo = jnp.where(col < 1, 1.0, 0.0).astype(jnp.float32)
    parts = []
    for i in range(N_MODULES_):
        sel = probs[:, i:i + 1] > 0.5
        parts.append(jnp.where(sel, hi, lo))
    return jnp.concatenate(parts, axis=-1)


def _softmax4(logits):
    m = jnp.max(logits, axis=-1, keepdims=True)
    e = jnp.exp(logits - m)
    return e / jnp.sum(e, axis=-1, keepdims=True)


def _prep_route_kernel(wr_ref, wg_ref, x_ref, m_ref, f_ref, c_acc):
    s = pl.program_id(0)

    @pl.when(s == 0)
    def _():
        c_acc[...] = jnp.zeros_like(c_acc)

    @pl.when(s < N_KC)
    def _():
        c_acc[...] += jax.lax.dot_general(
            wr_ref[...], wg_ref[...], (((1,), (0,)), ((), ())),
            preferred_element_type=jnp.float32)

    @pl.when(s >= N_KC)
    def _():
        logits = jax.lax.dot_general(
            x_ref[...], c_acc[...], (((1,), (1,)), ((), ())),
            preferred_element_type=jnp.float32)
        probs = _softmax4(logits)
        m_ref[...] = _masks32(probs, B_T)
        amb = jnp.any(jnp.abs(probs - 0.5) < WINDOW, axis=-1, keepdims=True)
        f_ref[...] = jnp.broadcast_to(amb, (B_T, N_EXPERTS_)).astype(jnp.int32)


def _prep_route_call():
    def _kidx(s):
        return jnp.minimum(s, N_KC - 1)

    def _tidx(s):
        return jnp.clip(s - N_KC, 0, N_BT - 1)

    return pl.pallas_call(
        _prep_route_kernel,
        grid=(N_KC + N_BT,),
        in_specs=[
            pl.BlockSpec((N_MODULES_, C_K), lambda s: (0, _kidx(s))),
            pl.BlockSpec((C_K, D_MODEL_), lambda s: (_kidx(s), 0)),
            pl.BlockSpec((B_T, D_MODEL_), lambda s: (_tidx(s), 0)),
        ],
        out_specs=[
            pl.BlockSpec((B_T, M_W), lambda s: (_tidx(s), 0)),
            pl.BlockSpec((B_T, N_EXPERTS_), lambda s: (_tidx(s), 0)),
        ],
        out_shape=[
            jax.ShapeDtypeStruct((B_, M_W), jnp.float32),
            jax.ShapeDtypeStruct((B_, N_EXPERTS_), jnp.int32),
        ],
        scratch_shapes=[pltpu.VMEM((N_MODULES_, D_MODEL_), jnp.float32)],
        compiler_params=pltpu.CompilerParams(
            dimension_semantics=("arbitrary",),
        ),
    )


def _faithful_kernel(p_ref, wg_ref, wr_ref, m_ref, acc_ref):
    kc = pl.program_id(1)

    @pl.when(kc == 0)
    def _():
        acc_ref[...] = jnp.zeros_like(acc_ref)

    gated = jax.lax.dot_general(
        p_ref[...], wg_ref[...], (((1,), (1,)), ((), ())),
        preferred_element_type=jnp.float32)
    acc_ref[...] += jax.lax.dot_general(
        gated, wr_ref[...], (((1,), (1,)), ((), ())),
        preferred_element_type=jnp.float32)

    @pl.when(kc == N_GC - 1)
    def _():
        probs = _softmax4(acc_ref[...])
        m_ref[...] = _masks32(probs, p_ref.shape[0])


def _faithful_call(n_rows):
    r_t = min(n_rows, B_T)
    return pl.pallas_call(
        _faithful_kernel,
        grid=(n_rows // r_t, N_GC),
        in_specs=[
            pl.BlockSpec((r_t, D_MODEL_), lambda i, k: (i, 0)),
            pl.BlockSpec((C_G, D_MODEL_), lambda i, k: (k, 0)),
            pl.BlockSpec((N_MODULES_, C_G), lambda i, k: (0, k)),
        ],
        out_specs=pl.BlockSpec((r_t, M_W), lambda i, k: (i, 0)),
        out_shape=jax.ShapeDtypeStruct((n_rows, M_W), jnp.float32),
        scratch_shapes=[pltpu.VMEM((r_t, N_MODULES_), jnp.float32)],
        compiler_params=pltpu.CompilerParams(
            dimension_semantics=("parallel", "arbitrary"),
        ),
    )


def kernel(pooled_hidden, Wg, Wr):
    masks, flags = _prep_route_call()(Wr, Wg, pooled_hidden)
    flag_row = flags[:, 0]
    n_amb = jnp.sum(flag_row)
    idx = jnp.nonzero(flag_row, size=CAP, fill_value=0)[0].astype(jnp.int32)

    def hybrid():
        x_amb = pooled_hidden[idx]
        rm = _faithful_call(CAP)(x_amb, Wg, Wr)
        return masks.at[idx].set(rm)

    def full_fallback():
        return _faithful_call(B_)(pooled_hidden, Wg, Wr)

    m = jax.lax.cond(n_amb <= CAP, hybrid, full_fallback)
    return tuple(m[:, i * N_EXPERTS_:(i + 1) * N_EXPERTS_]
                 for i in range(N_MODULES_))
